# baseline (speedup 1.0000x reference)
import functools

import jax
import jax.numpy as jnp
from jax.experimental import pallas as pl
from jax.experimental.pallas import tpu as pltpu

N = 4096
NFEAT = 512
NHID = 128
NHEADS = 4
NOUT = 128
ALPHA = 0.2

BR = 256
BC = 1024
BRP = 256


def _mm2_kernel(x_ref, wa_ref, ba_ref, wb_ref, bb_ref, oa_ref, ob_ref):
    x = x_ref[...]
    oa_ref[...] = (
        jnp.dot(x, wa_ref[...], preferred_element_type=jnp.float32) + ba_ref[...]
    ).astype(jnp.bfloat16)
    ob_ref[...] = (
        jnp.dot(x, wb_ref[...], preferred_element_type=jnp.float32) + bb_ref[...]
    )


def _mm2(x, wa, ba, wb, bb):
    n, k = x.shape
    ma = wa.shape[1]
    mb = wb.shape[1]
    grid = (n // BRP,)
    return pl.pallas_call(
        _mm2_kernel,
        grid=grid,
        in_specs=[
            pl.BlockSpec((BRP, k), lambda r: (r, 0)),
            pl.BlockSpec((k, ma), lambda r: (0, 0)),
            pl.BlockSpec((1, ma), lambda r: (0, 0)),
            pl.BlockSpec((k, mb), lambda r: (0, 0)),
            pl.BlockSpec((1, mb), lambda r: (0, 0)),
        ],
        out_specs=[
            pl.BlockSpec((BRP, ma), lambda r: (r, 0)),
            pl.BlockSpec((BRP, mb), lambda r: (r, 0)),
        ],
        out_shape=[
            jax.ShapeDtypeStruct((n, ma), jnp.bfloat16),
            jax.ShapeDtypeStruct((n, mb), jnp.float32),
        ],
    )(x, wa, ba, wb, bb)


def _stats_kernel(f_ref, rowv_ref, bv_ref):
    f = f_ref[...]
    f1 = f[:, 0:NHEADS]
    f2 = f[:, NHEADS : 2 * NHEADS]
    m2 = jnp.max(f2, axis=0, keepdims=True)
    t = f1 + m2
    m = jnp.maximum(t, ALPHA * t)
    rowv_ref[:, 0:NHEADS] = jnp.exp(t - m)
    rowv_ref[:, NHEADS : 2 * NHEADS] = jnp.exp(ALPHA * t - m)
    u = f2 - m2
    bv_ref[:, 0:NHEADS] = jnp.exp(u)
    bv_ref[:, NHEADS : 2 * NHEADS] = jnp.exp(ALPHA * u)


def _stats(f):
    return pl.pallas_call(
        _stats_kernel,
        out_shape=[
            jax.ShapeDtypeStruct((N, 2 * NHEADS), jnp.float32),
            jax.ShapeDtypeStruct((N, 2 * NHEADS), jnp.float32),
        ],
    )(f)


def _map_and_dot(adj, rowv_ref, colv_ref, v_ref, u_scr, p_out_ref, c):
    ps = []
    for i in range(NHEADS):
        ap = rowv_ref[:, i : i + 1]
        an = rowv_ref[:, NHEADS + i : NHEADS + i + 1]
        bp = colv_ref[i : i + 1, :]
        bn = colv_ref[NHEADS + i : NHEADS + i + 1, :]
        p = jnp.maximum(ap * bp, an * bn) * adj
        p_bf = p.astype(jnp.bfloat16)
        p_out_ref[i, :, :] = p_bf
        vblk = v_ref[pl.ds(c * BC, BC), i * NHID : (i + 1) * NHID]
        u_scr[:, i * NHID : (i + 1) * NHID] += jnp.dot(
            p_bf, vblk, preferred_element_type=jnp.float32
        )
        ps.append(p)
    return ps


def _norm_elu(u, den):
    cols = []
    for i in range(NHEADS):
        d = den[:, i : i + 1]
        ok = d > 0.0
        x = u[:, i * NHID : (i + 1) * NHID] / jnp.where(ok, d, 1.0)
        x = jnp.where(ok, x, 0.0)
        cols.append(jnp.where(x > 0.0, x, jnp.exp(x) - 1.0))
    return jnp.concatenate(cols, axis=1)


def _att1_kernel(adj_ref, rowv_ref, colv_ref, v_ref, out_ref, den_ref,
                 p_out_ref, u_scr, den_scr):
    c = pl.program_id(1)
    nc = pl.num_programs(1)

    @pl.when(c == 0)
    def _init():
        u_scr[...] = jnp.zeros_like(u_scr)
        den_scr[...] = jnp.zeros_like(den_scr)

    ps = _map_and_dot(adj_ref[...], rowv_ref, colv_ref, v_ref, u_scr,
                      p_out_ref, c)
    for i in range(NHEADS):
        den_scr[:, i : i + 1] += jnp.sum(ps[i], axis=1, keepdims=True)

    @pl.when(c == nc - 1)
    def _fin():
        den_ref[...] = den_scr[...]
        out_ref[...] = _norm_elu(u_scr[...], den_scr[...]).astype(jnp.bfloat16)


def _att1_pass(adjs, rowv, colv, v):
    grid = (N // BR, N // BC)
    return pl.pallas_call(
        _att1_kernel,
        grid=grid,
        in_specs=[
            pl.BlockSpec((BR, BC), lambda r, c: (r, c)),
            pl.BlockSpec((BR, 2 * NHEADS), lambda r, c: (r, 0)),
            pl.BlockSpec((2 * NHEADS, BC), lambda r, c: (0, c)),
            pl.BlockSpec((N, NHEADS * NHID), lambda r, c: (0, 0)),
        ],
        out_specs=[
            pl.BlockSpec((BR, NHEADS * NHID), lambda r, c: (r, 0)),
            pl.BlockSpec((BR, 2 * NHEADS), lambda r, c: (r, 0)),
            pl.BlockSpec((NHEADS, BR, BC), lambda r, c: (0, r, c)),
        ],
        out_shape=[
            jax.ShapeDtypeStruct((N, NHEADS * NHID), jnp.bfloat16),
            jax.ShapeDtypeStruct((N, 2 * NHEADS), jnp.float32),
            jax.ShapeDtypeStruct((NHEADS, N, N), jnp.bfloat16),
        ],
        scratch_shapes=[
            pltpu.VMEM((BR, NHEADS * NHID), jnp.float32),
            pltpu.VMEM((BR, 2 * NHEADS), jnp.float32),
        ],
        compiler_params=pltpu.CompilerParams(
            dimension_semantics=("arbitrary", "arbitrary"),
        ),
    )(adjs, rowv, colv, v)


def _att2_kernel(p_ref, v_ref, den_ref, wl_ref, bl_ref, out_ref, u_scr):
    c = pl.program_id(1)
    nc = pl.num_programs(1)

    @pl.when(c == 0)
    def _init():
        u_scr[...] = jnp.zeros_like(u_scr)

    for i in range(NHEADS):
        vblk = v_ref[pl.ds(c * BC, BC), i * NHID : (i + 1) * NHID]
        u_scr[:, i * NHID : (i + 1) * NHID] += jnp.dot(
            p_ref[i, :, :], vblk, preferred_element_type=jnp.float32
        )

    @pl.when(c == nc - 1)
    def _fin():
        x = _norm_elu(u_scr[...], den_ref[...])
        out_ref[...] = (
            jnp.dot(
                x.astype(jnp.bfloat16),
                wl_ref[...],
                preferred_element_type=jnp.float32,
            )
            + bl_ref[...]
        )


def _att2_pass(p, v, den, wl, bl):
    grid = (N // BR, N // BC)
    return pl.pallas_call(
        _att2_kernel,
        grid=grid,
        in_specs=[
            pl.BlockSpec((NHEADS, BR, BC), lambda r, c: (0, r, c)),
            pl.BlockSpec((N, NHEADS * NHID), lambda r, c: (0, 0)),
            pl.BlockSpec((BR, 2 * NHEADS), lambda r, c: (r, 0)),
            pl.BlockSpec((NHEADS * NHID, NOUT), lambda r, c: (0, 0)),
            pl.BlockSpec((1, NOUT), lambda r, c: (0, 0)),
        ],
        out_specs=pl.BlockSpec((BR, NOUT), lambda r, c: (r, 0)),
        out_shape=jax.ShapeDtypeStruct((N, NOUT), jnp.float32),
        scratch_shapes=[
            pltpu.VMEM((BR, NHEADS * NHID), jnp.float32),
        ],
        compiler_params=pltpu.CompilerParams(
            dimension_semantics=("arbitrary", "arbitrary"),
        ),
    )(p, v, den, wl, bl)


@jax.jit
def kernel(feat_data, adjs, fW_W, fW_b, a_src, a_dest, W0, b0, W1, b1, Wl, bl):
    w_src = jnp.einsum("hfk,hk->fh", fW_W, a_src)
    w_dst = jnp.einsum("hfk,hk->fh", fW_W, a_dest)
    wf = jnp.concatenate([w_src, w_dst], axis=1)
    cf = jnp.concatenate(
        [jnp.sum(fW_b * a_src, axis=1), jnp.sum(fW_b * a_dest, axis=1)]
    )[None, :]
    w0cat = jnp.concatenate(list(W0), axis=1)
    b0cat = jnp.concatenate(list(b0))[None, :]
    w1cat = jnp.concatenate(list(W1), axis=1)
    b1cat = jnp.concatenate(list(b1))[None, :]

    v0, f = _mm2(feat_data, w0cat, b0cat, wf, cf)
    rowv, bv = _stats(f)
    colv = bv.T

    x1, den, p = _att1_pass(adjs, rowv, colv, v0)
    v1, _ = _mm2(x1, w1cat, b1cat, wf, cf)
    out = _att2_pass(p, v1, den, Wl, bl[None, :])
    return out


# speedup vs baseline: 1.2362x; 1.2362x over previous
import jax
import jax.numpy as jnp
from jax.experimental import pallas as pl
from jax.experimental.pallas import tpu as pltpu

N = 4096
NFEAT = 512
NHID = 128
NHEADS = 4
NOUT = 128
ALPHA = 0.2

BR = 256
BC = 2048


def _stats_kernel(feat_ref, wf_ref, cf_ref, rowv_ref, bv_ref):
    f = (
        jnp.dot(feat_ref[...], wf_ref[...], preferred_element_type=jnp.float32)
        + cf_ref[...]
    )
    f1 = f[:, 0:NHEADS]
    f2 = f[:, NHEADS : 2 * NHEADS]
    m2 = jnp.max(f2, axis=0, keepdims=True)
    t = f1 + m2
    m = jnp.maximum(t, ALPHA * t)
    rowv_ref[:, 0:NHEADS] = jnp.exp(t - m)
    rowv_ref[:, NHEADS : 2 * NHEADS] = jnp.exp(ALPHA * t - m)
    u = f2 - m2
    bv_ref[:, 0:NHEADS] = jnp.exp(u)
    bv_ref[:, NHEADS : 2 * NHEADS] = jnp.exp(ALPHA * u)


def _stats(feat, wf, cf):
    return pl.pallas_call(
        _stats_kernel,
        out_shape=[
            jax.ShapeDtypeStruct((N, 2 * NHEADS), jnp.float32),
            jax.ShapeDtypeStruct((N, 2 * NHEADS), jnp.float32),
        ],
    )(feat, wf, cf)


def _map_and_dot(adj, rowv_ref, colv_ref, v_scr, u_scr, c):
    ps = []
    for i in range(NHEADS):
        ap = rowv_ref[:, i : i + 1]
        an = rowv_ref[:, NHEADS + i : NHEADS + i + 1]
        bp = colv_ref[i : i + 1, :]
        bn = colv_ref[NHEADS + i : NHEADS + i + 1, :]
        p = jnp.maximum(ap * bp, an * bn) * adj
        vblk = v_scr[pl.ds(c * BC, BC), i * NHID : (i + 1) * NHID]
        u_scr[:, i * NHID : (i + 1) * NHID] += jnp.dot(
            p.astype(jnp.bfloat16), vblk, preferred_element_type=jnp.float32
        )
        ps.append(p)
    return ps


def _norm_elu(u, den):
    cols = []
    for i in range(NHEADS):
        d = den[:, i : i + 1]
        ok = d > 0.0
        x = u[:, i * NHID : (i + 1) * NHID] / jnp.where(ok, d, 1.0)
        x = jnp.where(ok, x, 0.0)
        cols.append(jnp.where(x > 0.0, x, jnp.exp(x) - 1.0))
    return jnp.concatenate(cols, axis=1)


def _att1_kernel(adj_ref, rowv_ref, colv_ref, x_ref, w_ref, b_ref,
                 out_ref, den_ref, u_scr, den_scr, v_scr):
    r = pl.program_id(0)
    c = pl.program_id(1)
    nc = pl.num_programs(1)

    @pl.when(r == 0)
    def _make_v():
        xblk = x_ref[pl.ds(c * BC, BC), :]
        v_scr[pl.ds(c * BC, BC), :] = (
            jnp.dot(xblk, w_ref[...], preferred_element_type=jnp.float32)
            + b_ref[...]
        ).astype(jnp.bfloat16)

    @pl.when(c == 0)
    def _init():
        u_scr[...] = jnp.zeros_like(u_scr)
        den_scr[...] = jnp.zeros_like(den_scr)

    ps = _map_and_dot(adj_ref[...], rowv_ref, colv_ref, v_scr, u_scr, c)
    for i in range(NHEADS):
        den_scr[:, i : i + 1] += jnp.sum(ps[i], axis=1, keepdims=True)

    @pl.when(c == nc - 1)
    def _fin():
        den_ref[...] = den_scr[...]
        out_ref[...] = _norm_elu(u_scr[...], den_scr[...]).astype(jnp.bfloat16)


def _att1_pass(adjs, rowv, colv, feat_bf, w0cat, b0cat):
    grid = (N // BR, N // BC)
    return pl.pallas_call(
        _att1_kernel,
        grid=grid,
        in_specs=[
            pl.BlockSpec((BR, BC), lambda r, c: (r, c)),
            pl.BlockSpec((BR, 2 * NHEADS), lambda r, c: (r, 0)),
            pl.BlockSpec((2 * NHEADS, BC), lambda r, c: (0, c)),
            pl.BlockSpec((N, NFEAT), lambda r, c: (0, 0)),
            pl.BlockSpec((NFEAT, NHEADS * NHID), lambda r, c: (0, 0)),
            pl.BlockSpec((1, NHEADS * NHID), lambda r, c: (0, 0)),
        ],
        out_specs=[
            pl.BlockSpec((BR, NHEADS * NHID), lambda r, c: (r, 0)),
            pl.BlockSpec((BR, 2 * NHEADS), lambda r, c: (r, 0)),
        ],
        out_shape=[
            jax.ShapeDtypeStruct((N, NHEADS * NHID), jnp.bfloat16),
            jax.ShapeDtypeStruct((N, 2 * NHEADS), jnp.float32),
        ],
        scratch_shapes=[
            pltpu.VMEM((BR, NHEADS * NHID), jnp.float32),
            pltpu.VMEM((BR, 2 * NHEADS), jnp.float32),
            pltpu.VMEM((N, NHEADS * NHID), jnp.bfloat16),
        ],
        compiler_params=pltpu.CompilerParams(
            dimension_semantics=("arbitrary", "arbitrary"),
        ),
    )(adjs, rowv, colv, feat_bf, w0cat, b0cat)


def _att2_kernel(adj_ref, rowv_ref, colv_ref, x_ref, w_ref, b_ref, den_ref,
                 wl_ref, bl_ref, out_ref, u_scr, v_scr):
    r = pl.program_id(0)
    c = pl.program_id(1)
    nc = pl.num_programs(1)

    @pl.when(r == 0)
    def _make_v():
        xblk = x_ref[pl.ds(c * BC, BC), :]
        v_scr[pl.ds(c * BC, BC), :] = (
            jnp.dot(xblk, w_ref[...], preferred_element_type=jnp.float32)
            + b_ref[...]
        ).astype(jnp.bfloat16)

    @pl.when(c == 0)
    def _init():
        u_scr[...] = jnp.zeros_like(u_scr)

    _map_and_dot(adj_ref[...], rowv_ref, colv_ref, v_scr, u_scr, c)

    @pl.when(c == nc - 1)
    def _fin():
        x = _norm_elu(u_scr[...], den_ref[...])
        out_ref[...] = (
            jnp.dot(
                x.astype(jnp.bfloat16),
                wl_ref[...],
                preferred_element_type=jnp.float32,
            )
            + bl_ref[...]
        )


def _att2_pass(adjs, rowv, colv, x1, w1cat, b1cat, den, wl, bl):
    grid = (N // BR, N // BC)
    return pl.pallas_call(
        _att2_kernel,
        grid=grid,
        in_specs=[
            pl.BlockSpec((BR, BC), lambda r, c: (r, c)),
            pl.BlockSpec((BR, 2 * NHEADS), lambda r, c: (r, 0)),
            pl.BlockSpec((2 * NHEADS, BC), lambda r, c: (0, c)),
            pl.BlockSpec((N, NHEADS * NHID), lambda r, c: (0, 0)),
            pl.BlockSpec((NHEADS * NHID, NHEADS * NHID), lambda r, c: (0, 0)),
            pl.BlockSpec((1, NHEADS * NHID), lambda r, c: (0, 0)),
            pl.BlockSpec((BR, 2 * NHEADS), lambda r, c: (r, 0)),
            pl.BlockSpec((NHEADS * NHID, NOUT), lambda r, c: (0, 0)),
            pl.BlockSpec((1, NOUT), lambda r, c: (0, 0)),
        ],
        out_specs=pl.BlockSpec((BR, NOUT), lambda r, c: (r, 0)),
        out_shape=jax.ShapeDtypeStruct((N, NOUT), jnp.float32),
        scratch_shapes=[
            pltpu.VMEM((BR, NHEADS * NHID), jnp.float32),
            pltpu.VMEM((N, NHEADS * NHID), jnp.bfloat16),
        ],
        compiler_params=pltpu.CompilerParams(
            dimension_semantics=("arbitrary", "arbitrary"),
        ),
    )(adjs, rowv, colv, x1, w1cat, b1cat, den, wl, bl)


@jax.jit
def kernel(feat_data, adjs, fW_W, fW_b, a_src, a_dest, W0, b0, W1, b1, Wl, bl):
    w_src = jnp.einsum("hfk,hk->fh", fW_W, a_src)
    w_dst = jnp.einsum("hfk,hk->fh", fW_W, a_dest)
    wf = jnp.concatenate([w_src, w_dst], axis=1)
    cf = jnp.concatenate(
        [jnp.sum(fW_b * a_src, axis=1), jnp.sum(fW_b * a_dest, axis=1)]
    )[None, :]
    w0cat = jnp.concatenate(list(W0), axis=1).astype(jnp.bfloat16)
    b0cat = jnp.concatenate(list(b0))[None, :]
    w1cat = jnp.concatenate(list(W1), axis=1).astype(jnp.bfloat16)
    b1cat = jnp.concatenate(list(b1))[None, :]
    feat_bf = feat_data.astype(jnp.bfloat16)

    rowv, bv = _stats(feat_data, wf, cf)
    colv = bv.T

    x1, den = _att1_pass(adjs, rowv, colv, feat_bf, w0cat, b0cat)
    out = _att2_pass(adjs, rowv, colv, x1, w1cat, b1cat, den,
                     Wl.astype(jnp.bfloat16), bl[None, :])
    return out


# speedup vs baseline: 1.3504x; 1.0924x over previous
import jax
import jax.numpy as jnp
from jax.experimental import pallas as pl
from jax.experimental.pallas import tpu as pltpu

N = 4096
NFEAT = 512
NHID = 128
NHEADS = 4
NOUT = 128
ALPHA = 0.2

BR = 256


def _stats_kernel(feat_ref, wf_ref, cf_ref, rowv_ref, bv_ref):
    f = (
        jnp.dot(feat_ref[...], wf_ref[...], preferred_element_type=jnp.float32)
        + cf_ref[...]
    )
    f1 = f[:, 0:NHEADS]
    f2 = f[:, NHEADS : 2 * NHEADS]
    m2 = jnp.max(f2, axis=0, keepdims=True)
    t = f1 + m2
    m = jnp.maximum(t, ALPHA * t)
    rowv_ref[:, 0:NHEADS] = jnp.exp(t - m)
    rowv_ref[:, NHEADS : 2 * NHEADS] = jnp.exp(ALPHA * t - m)
    u = f2 - m2
    bv_ref[:, 0:NHEADS] = jnp.exp(u)
    bv_ref[:, NHEADS : 2 * NHEADS] = jnp.exp(ALPHA * u)


def _stats(feat, wf, cf):
    return pl.pallas_call(
        _stats_kernel,
        out_shape=[
            jax.ShapeDtypeStruct((N, 2 * NHEADS), jnp.float32),
            jax.ShapeDtypeStruct((N, 2 * NHEADS), jnp.float32),
        ],
    )(feat, wf, cf)


def _heads(adj, rowv_ref, colv_ref, v_scr):
    us = []
    dens = []
    for i in range(NHEADS):
        ap = rowv_ref[:, i : i + 1]
        an = rowv_ref[:, NHEADS + i : NHEADS + i + 1]
        bp = colv_ref[i : i + 1, :]
        bn = colv_ref[NHEADS + i : NHEADS + i + 1, :]
        p = jnp.maximum(ap * bp, an * bn) * adj
        us.append(
            jnp.dot(
                p.astype(jnp.bfloat16),
                v_scr[:, i * NHID : (i + 1) * NHID],
                preferred_element_type=jnp.float32,
            )
        )
        dens.append(jnp.sum(p, axis=1, keepdims=True))
    return us, dens


def _norm_elu(us, dens):
    cols = []
    for i in range(NHEADS):
        d = dens[i]
        ok = d > 0.0
        x = us[i] / jnp.where(ok, d, 1.0)
        x = jnp.where(ok, x, 0.0)
        cols.append(jnp.where(x > 0.0, x, jnp.exp(x) - 1.0))
    return jnp.concatenate(cols, axis=1)


def _att1_kernel(adj_ref, rowv_ref, colv_ref, x_ref, w_ref, b_ref,
                 out_ref, den_ref, v_scr):
    r = pl.program_id(0)

    @pl.when(r == 0)
    def _make_v():
        v_scr[...] = (
            jnp.dot(x_ref[...], w_ref[...], preferred_element_type=jnp.float32)
            + b_ref[...]
        ).astype(jnp.bfloat16)

    us, dens = _heads(adj_ref[...], rowv_ref, colv_ref, v_scr)
    den_ref[...] = jnp.concatenate(dens, axis=1)
    out_ref[...] = _norm_elu(us, dens).astype(jnp.bfloat16)


def _att1_pass(adjs, rowv, colv, feat_bf, w0cat, b0cat):
    grid = (N // BR,)
    return pl.pallas_call(
        _att1_kernel,
        grid=grid,
        in_specs=[
            pl.BlockSpec((BR, N), lambda r: (r, 0)),
            pl.BlockSpec((BR, 2 * NHEADS), lambda r: (r, 0)),
            pl.BlockSpec((2 * NHEADS, N), lambda r: (0, 0)),
            pl.BlockSpec((N, NFEAT), lambda r: (0, 0)),
            pl.BlockSpec((NFEAT, NHEADS * NHID), lambda r: (0, 0)),
            pl.BlockSpec((1, NHEADS * NHID), lambda r: (0, 0)),
        ],
        out_specs=[
            pl.BlockSpec((BR, NHEADS * NHID), lambda r: (r, 0)),
            pl.BlockSpec((BR, NHEADS), lambda r: (r, 0)),
        ],
        out_shape=[
            jax.ShapeDtypeStruct((N, NHEADS * NHID), jnp.bfloat16),
            jax.ShapeDtypeStruct((N, NHEADS), jnp.float32),
        ],
        scratch_shapes=[
            pltpu.VMEM((N, NHEADS * NHID), jnp.bfloat16),
        ],
        compiler_params=pltpu.CompilerParams(
            dimension_semantics=("arbitrary",),
        ),
    )(adjs, rowv, colv, feat_bf, w0cat, b0cat)


def _att2_kernel(adj_ref, rowv_ref, colv_ref, x_ref, w_ref, b_ref, den_ref,
                 wl_ref, bl_ref, out_ref, v_scr):
    r = pl.program_id(0)

    @pl.when(r == 0)
    def _make_v():
        v_scr[...] = (
            jnp.dot(x_ref[...], w_ref[...], preferred_element_type=jnp.float32)
            + b_ref[...]
        ).astype(jnp.bfloat16)

    us, _ = _heads(adj_ref[...], rowv_ref, colv_ref, v_scr)
    den = den_ref[...]
    dens = [den[:, i : i + 1] for i in range(NHEADS)]
    x = _norm_elu(us, dens)
    out_ref[...] = (
        jnp.dot(
            x.astype(jnp.bfloat16), wl_ref[...],
            preferred_element_type=jnp.float32,
        )
        + bl_ref[...]
    )


def _att2_pass(adjs, rowv, colv, x1, w1cat, b1cat, den, wl, bl):
    grid = (N // BR,)
    return pl.pallas_call(
        _att2_kernel,
        grid=grid,
        in_specs=[
            pl.BlockSpec((BR, N), lambda r: (r, 0)),
            pl.BlockSpec((BR, 2 * NHEADS), lambda r: (r, 0)),
            pl.BlockSpec((2 * NHEADS, N), lambda r: (0, 0)),
            pl.BlockSpec((N, NHEADS * NHID), lambda r: (0, 0)),
            pl.BlockSpec((NHEADS * NHID, NHEADS * NHID), lambda r: (0, 0)),
            pl.BlockSpec((1, NHEADS * NHID), lambda r: (0, 0)),
            pl.BlockSpec((BR, NHEADS), lambda r: (r, 0)),
            pl.BlockSpec((NHEADS * NHID, NOUT), lambda r: (0, 0)),
            pl.BlockSpec((1, NOUT), lambda r: (0, 0)),
        ],
        out_specs=pl.BlockSpec((BR, NOUT), lambda r: (r, 0)),
        out_shape=jax.ShapeDtypeStruct((N, NOUT), jnp.float32),
        scratch_shapes=[
            pltpu.VMEM((N, NHEADS * NHID), jnp.bfloat16),
        ],
        compiler_params=pltpu.CompilerParams(
            dimension_semantics=("arbitrary",),
        ),
    )(adjs, rowv, colv, x1, w1cat, b1cat, den, wl, bl)


@jax.jit
def kernel(feat_data, adjs, fW_W, fW_b, a_src, a_dest, W0, b0, W1, b1, Wl, bl):
    w_src = jnp.einsum("hfk,hk->fh", fW_W, a_src)
    w_dst = jnp.einsum("hfk,hk->fh", fW_W, a_dest)
    wf = jnp.concatenate([w_src, w_dst], axis=1)
    cf = jnp.concatenate(
        [jnp.sum(fW_b * a_src, axis=1), jnp.sum(fW_b * a_dest, axis=1)]
    )[None, :]
    w0cat = jnp.concatenate(list(W0), axis=1).astype(jnp.bfloat16)
    b0cat = jnp.concatenate(list(b0))[None, :]
    w1cat = jnp.concatenate(list(W1), axis=1).astype(jnp.bfloat16)
    b1cat = jnp.concatenate(list(b1))[None, :]
    feat_bf = feat_data.astype(jnp.bfloat16)

    rowv, bv = _stats(feat_data, wf, cf)
    colv = bv.T

    x1, den = _att1_pass(adjs, rowv, colv, feat_bf, w0cat, b0cat)
    out = _att2_pass(adjs, rowv, colv, x1, w1cat, b1cat, den,
                     Wl.astype(jnp.bfloat16), bl[None, :])
    return out


# speedup vs baseline: 1.4291x; 1.0583x over previous
import jax
import jax.numpy as jnp
from jax.experimental import pallas as pl
from jax.experimental.pallas import tpu as pltpu

N = 4096
NFEAT = 512
NHID = 128
NHEADS = 4
NOUT = 128
ALPHA = 0.2

BR = 256


def _stats_kernel(feat_ref, wf_ref, cf_ref, rowv_ref, bv_ref):
    f = (
        jnp.dot(feat_ref[...], wf_ref[...], preferred_element_type=jnp.float32)
        + cf_ref[...]
    )
    f1 = f[:, 0:NHEADS]
    f2 = f[:, NHEADS : 2 * NHEADS]
    m2 = jnp.max(f2, axis=0, keepdims=True)
    t = f1 + m2
    m = jnp.maximum(t, ALPHA * t)
    rowv_ref[:, 0:NHEADS] = jnp.exp(t - m)
    rowv_ref[:, NHEADS : 2 * NHEADS] = jnp.exp(ALPHA * t - m)
    u = f2 - m2
    bv_ref[:, 0:NHEADS] = jnp.exp(u)
    bv_ref[:, NHEADS : 2 * NHEADS] = jnp.exp(ALPHA * u)


def _stats(feat, wf, cf):
    return pl.pallas_call(
        _stats_kernel,
        out_shape=[
            jax.ShapeDtypeStruct((N, 2 * NHEADS), jnp.float32),
            jax.ShapeDtypeStruct((N, 2 * NHEADS), jnp.float32),
        ],
    )(feat, wf, cf)


CHUNK = 512


def _heads(adj_ref, rowv_ref, colv_ref, v_scr):
    us = [jnp.zeros((BR, NHID), jnp.float32) for _ in range(NHEADS)]
    dens = [jnp.zeros((BR, 1), jnp.float32) for _ in range(NHEADS)]
    for k in range(N // CHUNK):
        adj = adj_ref[:, k * CHUNK : (k + 1) * CHUNK]
        for i in range(NHEADS):
            ap = rowv_ref[:, i : i + 1]
            an = rowv_ref[:, NHEADS + i : NHEADS + i + 1]
            bp = colv_ref[i : i + 1, k * CHUNK : (k + 1) * CHUNK]
            bn = colv_ref[NHEADS + i : NHEADS + i + 1,
                          k * CHUNK : (k + 1) * CHUNK]
            p = jnp.maximum(ap * bp, an * bn) * adj
            us[i] += jnp.dot(
                p.astype(jnp.bfloat16),
                v_scr[k * CHUNK : (k + 1) * CHUNK, i * NHID : (i + 1) * NHID],
                preferred_element_type=jnp.float32,
            )
            dens[i] += jnp.sum(p, axis=1, keepdims=True)
    return us, dens


def _norm_elu(us, dens):
    cols = []
    for i in range(NHEADS):
        d = dens[i]
        ok = d > 0.0
        x = us[i] / jnp.where(ok, d, 1.0)
        x = jnp.where(ok, x, 0.0)
        cols.append(jnp.where(x > 0.0, x, jnp.exp(x) - 1.0))
    return jnp.concatenate(cols, axis=1)


def _att1_kernel(adj_ref, rowv_ref, colv_ref, x_ref, w_ref, b_ref,
                 out_ref, den_ref, v_scr):
    r = pl.program_id(0)

    @pl.when(r == 0)
    def _make_v():
        v_scr[...] = (
            jnp.dot(x_ref[...], w_ref[...], preferred_element_type=jnp.float32)
            + b_ref[...]
        ).astype(jnp.bfloat16)

    us, dens = _heads(adj_ref[...], rowv_ref, colv_ref, v_scr)
    den_ref[...] = jnp.concatenate(dens, axis=1)
    out_ref[...] = _norm_elu(us, dens).astype(jnp.bfloat16)


def _att1_pass(adjs, rowv, colv, feat_bf, w0cat, b0cat):
    grid = (N // BR,)
    return pl.pallas_call(
        _att1_kernel,
        grid=grid,
        in_specs=[
            pl.BlockSpec((BR, N), lambda r: (r, 0)),
            pl.BlockSpec((BR, 2 * NHEADS), lambda r: (r, 0)),
            pl.BlockSpec((2 * NHEADS, N), lambda r: (0, 0)),
            pl.BlockSpec((N, NFEAT), lambda r: (0, 0)),
            pl.BlockSpec((NFEAT, NHEADS * NHID), lambda r: (0, 0)),
            pl.BlockSpec((1, NHEADS * NHID), lambda r: (0, 0)),
        ],
        out_specs=[
            pl.BlockSpec((BR, NHEADS * NHID), lambda r: (r, 0)),
            pl.BlockSpec((BR, NHEADS), lambda r: (r, 0)),
        ],
        out_shape=[
            jax.ShapeDtypeStruct((N, NHEADS * NHID), jnp.bfloat16),
            jax.ShapeDtypeStruct((N, NHEADS), jnp.float32),
        ],
        scratch_shapes=[
            pltpu.VMEM((N, NHEADS * NHID), jnp.bfloat16),
        ],
        compiler_params=pltpu.CompilerParams(
            dimension_semantics=("arbitrary",),
        ),
    )(adjs, rowv, colv, feat_bf, w0cat, b0cat)


def _att2_kernel(adj_ref, rowv_ref, colv_ref, x_ref, w_ref, b_ref, den_ref,
                 wl_ref, bl_ref, out_ref, v_scr):
    r = pl.program_id(0)

    @pl.when(r == 0)
    def _make_v():
        v_scr[...] = (
            jnp.dot(x_ref[...], w_ref[...], preferred_element_type=jnp.float32)
            + b_ref[...]
        ).astype(jnp.bfloat16)

    us, _ = _heads(adj_ref[...], rowv_ref, colv_ref, v_scr)
    den = den_ref[...]
    dens = [den[:, i : i + 1] for i in range(NHEADS)]
    x = _norm_elu(us, dens)
    out_ref[...] = (
        jnp.dot(
            x.astype(jnp.bfloat16), wl_ref[...],
            preferred_element_type=jnp.float32,
        )
        + bl_ref[...]
    )


def _att2_pass(adjs, rowv, colv, x1, w1cat, b1cat, den, wl, bl):
    grid = (N // BR,)
    return pl.pallas_call(
        _att2_kernel,
        grid=grid,
        in_specs=[
            pl.BlockSpec((BR, N), lambda r: (r, 0)),
            pl.BlockSpec((BR, 2 * NHEADS), lambda r: (r, 0)),
            pl.BlockSpec((2 * NHEADS, N), lambda r: (0, 0)),
            pl.BlockSpec((N, NHEADS * NHID), lambda r: (0, 0)),
            pl.BlockSpec((NHEADS * NHID, NHEADS * NHID), lambda r: (0, 0)),
            pl.BlockSpec((1, NHEADS * NHID), lambda r: (0, 0)),
            pl.BlockSpec((BR, NHEADS), lambda r: (r, 0)),
            pl.BlockSpec((NHEADS * NHID, NOUT), lambda r: (0, 0)),
            pl.BlockSpec((1, NOUT), lambda r: (0, 0)),
        ],
        out_specs=pl.BlockSpec((BR, NOUT), lambda r: (r, 0)),
        out_shape=jax.ShapeDtypeStruct((N, NOUT), jnp.float32),
        scratch_shapes=[
            pltpu.VMEM((N, NHEADS * NHID), jnp.bfloat16),
        ],
        compiler_params=pltpu.CompilerParams(
            dimension_semantics=("arbitrary",),
        ),
    )(adjs, rowv, colv, x1, w1cat, b1cat, den, wl, bl)


@jax.jit
def kernel(feat_data, adjs, fW_W, fW_b, a_src, a_dest, W0, b0, W1, b1, Wl, bl):
    w_src = jnp.einsum("hfk,hk->fh", fW_W, a_src)
    w_dst = jnp.einsum("hfk,hk->fh", fW_W, a_dest)
    wf = jnp.concatenate([w_src, w_dst], axis=1)
    cf = jnp.concatenate(
        [jnp.sum(fW_b * a_src, axis=1), jnp.sum(fW_b * a_dest, axis=1)]
    )[None, :]
    w0cat = jnp.concatenate(list(W0), axis=1).astype(jnp.bfloat16)
    b0cat = jnp.concatenate(list(b0))[None, :]
    w1cat = jnp.concatenate(list(W1), axis=1).astype(jnp.bfloat16)
    b1cat = jnp.concatenate(list(b1))[None, :]
    feat_bf = feat_data.astype(jnp.bfloat16)

    rowv, bv = _stats(feat_data, wf, cf)
    colv = bv.T

    x1, den = _att1_pass(adjs, rowv, colv, feat_bf, w0cat, b0cat)
    out = _att2_pass(adjs, rowv, colv, x1, w1cat, b1cat, den,
                     Wl.astype(jnp.bfloat16), bl[None, :])
    return out
